# baseline (speedup 1.0000x reference)
import functools

import jax
import jax.numpy as jnp
from jax import lax
from jax.experimental import pallas as pl
from jax.experimental.pallas import tpu as pltpu
from jax.experimental.pallas import tpu_sc as plsc

NC = 2
NS = 16
L = 16
B = 8
HW = 147456
GROUP = 4
TPT = HW // GROUP
CH = 4096
NCH = TPT // CH
NBIN = 1024
HLEN = L * NBIN


def _extract(v, j):
    return jnp.sum(jnp.where(lax.iota(jnp.int32, L) == j, v, jnp.zeros_like(v)))


def _put(v, j, val):
    return jnp.where(lax.iota(jnp.int32, L) == j, val, v)


def _zero_hists(cnt_h, sum_h):
    @plsc.parallel_loop(0, NBIN, unroll=8)
    def zbody(i):
        cnt_h[pl.ds(i * L, L)] = jnp.zeros((L,), jnp.int32)
        sum_h[pl.ds(i * L, L)] = jnp.zeros((L,), jnp.float32)


def _zero_cnt(cnt_h):
    @plsc.parallel_loop(0, NBIN, unroll=8)
    def zbody(i):
        cnt_h[pl.ds(i * L, L)] = jnp.zeros((L,), jnp.int32)


_SENT = 0x7FFFFFFF


def _hist_pass1(img, gidx, pred_hbm, targ_hbm, mask_hbm, bufs, sems, eb_c,
                cnt_h, sum_h):
    lane_off = lax.iota(jnp.int32, L) * NBIN
    ones = jnp.ones((L,), jnp.int32)
    base = gidx * TPT

    def issue(c):
        slot = c % 2
        off = base + c * CH
        bp, bt, bm = bufs[slot]
        sem = sems[slot]
        return (pltpu.async_copy(pred_hbm.at[img, pl.ds(off, CH)], bp, sem),
                pltpu.async_copy(targ_hbm.at[img, pl.ds(off, CH)], bt, sem),
                pltpu.async_copy(mask_hbm.at[img, pl.ds(off, CH)], bm, sem))

    pend = issue(0)
    for c in range(NCH):
        for d_ in pend:
            d_.wait()
        if c + 1 < NCH:
            pend = issue(c + 1)
        bp, bt, bm = bufs[c % 2]
        cbase = c * CH

        @plsc.parallel_loop(0, CH // L, unroll=4)
        def vec_body(j):
            s = pl.ds(j * L, L)
            err = jnp.abs(bp[s] - bt[s])
            valid = bm[s] > 0
            eb = lax.bitcast_convert_type(err, jnp.int32)
            eb_c[pl.ds(cbase + j * L, L)] = jnp.where(valid, eb,
                                                      jnp.int32(_SENT))
            idx = lane_off + lax.shift_right_logical(eb, 21)
            plsc.addupdate_scatter(cnt_h, [idx], ones, mask=valid)
            plsc.addupdate_scatter(sum_h, [idx], err, mask=valid)


def _hist_pass2(eb_c, cnt_h, bstar):
    lane_off = lax.iota(jnp.int32, L) * NBIN
    ones = jnp.ones((L,), jnp.int32)

    @plsc.parallel_loop(0, TPT // L, unroll=4)
    def vec_body(j):
        eb = eb_c[pl.ds(j * L, L)]
        valid = lax.shift_right_logical(eb, 21) == bstar
        idx = lane_off + jnp.bitwise_and(lax.shift_right_logical(eb, 11),
                                         jnp.int32(NBIN - 1))
        plsc.addupdate_scatter(cnt_h, [idx], ones, mask=valid)


def _lane_reduce(cnt_h, sum_h, cnt_m, sum_m):
    @plsc.parallel_loop(0, NBIN // L, unroll=2)
    def rbody(c):
        ac = jnp.zeros((L,), jnp.int32)
        asm = jnp.zeros((L,), jnp.float32)
        for r in range(L):
            s = pl.ds(r * NBIN + c * L, L)
            ac = ac + cnt_h[s]
            asm = asm + sum_h[s]
        cnt_m[pl.ds(c * L, L)] = ac
        sum_m[pl.ds(c * L, L)] = asm


def _lane_reduce_cnt(cnt_h, cnt_m):
    @plsc.parallel_loop(0, NBIN // L, unroll=2)
    def rbody(c):
        ac = jnp.zeros((L,), jnp.int32)
        for r in range(L):
            ac = ac + cnt_h[pl.ds(r * NBIN + c * L, L)]
        cnt_m[pl.ds(c * L, L)] = ac


def _merge_group_cnt(sid, cnt_sh, cnt_h, cnt_m):
    for j in range(GROUP):
        pltpu.sync_copy(cnt_sh.at[sid + j], cnt_h.at[pl.ds(j * NBIN, NBIN)])

    @plsc.parallel_loop(0, NBIN // L, unroll=4)
    def mbody(c):
        ac = jnp.zeros((L,), jnp.int32)
        for j in range(GROUP):
            ac = ac + cnt_h[pl.ds(j * NBIN + c * L, L)]
        cnt_m[pl.ds(c * L, L)] = ac


def _scan_select_mid(cnt_m, kk, bstar):
    lane = lax.iota(jnp.int32, L)
    hi = lax.shift_left(bstar, 21) + jnp.int32(0x400)

    def sbody(c, carry):
        run, nb, cb, sb = carry
        v = cnt_m[pl.ds(c * L, L)]
        bits = hi + lax.shift_left(c * L + lane, 11)
        mid = lax.bitcast_convert_type(bits, jnp.float32)
        cum = plsc.cumsum(v) + run
        m = cum < kk
        nb = nb + jnp.where(m, 1, 0).astype(jnp.int32)
        cb = cb + jnp.where(m, v, 0)
        sb = sb + jnp.where(m, v.astype(jnp.float32) * mid, jnp.float32(0.0))
        return run + jnp.sum(v), nb, cb, sb

    z_i = jnp.zeros((L,), jnp.int32)
    z_f = jnp.zeros((L,), jnp.float32)
    _, nb, cb, sb = lax.fori_loop(0, NBIN // L, sbody,
                                  (jnp.int32(0), z_i, z_i, z_f))
    return jnp.sum(nb), jnp.sum(cb), jnp.sum(sb)


def _merge_group(sid, cnt_sh, sum_sh, cnt_h, sum_h, cnt_m, sum_m):
    for j in range(GROUP):
        pltpu.sync_copy(cnt_sh.at[sid + j], cnt_h.at[pl.ds(j * NBIN, NBIN)])
        pltpu.sync_copy(sum_sh.at[sid + j], sum_h.at[pl.ds(j * NBIN, NBIN)])

    @plsc.parallel_loop(0, NBIN // L, unroll=4)
    def mbody(c):
        ac = jnp.zeros((L,), jnp.int32)
        asm = jnp.zeros((L,), jnp.float32)
        for j in range(GROUP):
            s = pl.ds(j * NBIN + c * L, L)
            ac = ac + cnt_h[s]
            asm = asm + sum_h[s]
        cnt_m[pl.ds(c * L, L)] = ac
        sum_m[pl.ds(c * L, L)] = asm


def _scan_select(cnt_m, sum_m, kk):
    def sbody(c, carry):
        run, nb, cb, sb = carry
        v = cnt_m[pl.ds(c * L, L)]
        sv = sum_m[pl.ds(c * L, L)]
        cum = plsc.cumsum(v) + run
        m = cum < kk
        nb = nb + jnp.where(m, 1, 0).astype(jnp.int32)
        cb = cb + jnp.where(m, v, 0)
        sb = sb + jnp.where(m, sv, jnp.float32(0.0))
        return run + jnp.sum(v), nb, cb, sb

    z_i = jnp.zeros((L,), jnp.int32)
    z_f = jnp.zeros((L,), jnp.float32)
    _, nb, cb, sb = lax.fori_loop(0, NBIN // L, sbody,
                                  (jnp.int32(0), z_i, z_i, z_f))
    return jnp.sum(nb), jnp.sum(cb), jnp.sum(sb)


def _bin_at(cnt_m, sum_m, b):
    lane = lax.iota(jnp.int32, L)

    def gbody(c, carry):
        ac, asm = carry
        gidx = c * L + lane
        sel = gidx == b
        ac = ac + jnp.where(sel, cnt_m[pl.ds(c * L, L)], 0)
        asm = asm + jnp.where(sel, sum_m[pl.ds(c * L, L)], jnp.float32(0.0))
        return ac, asm

    ac, asm = lax.fori_loop(0, NBIN // L, gbody,
                            (jnp.zeros((L,), jnp.int32),
                             jnp.zeros((L,), jnp.float32)))
    return jnp.sum(ac), jnp.sum(asm)


def _sc_body(pred_hbm, targ_hbm, mask_hbm, out_hbm,
             bp0, bt0, bm0, bp1, bt1, bm1, sem0, sem1, eb_c,
             cnt_h, sum_h, cnt_m, sum_m, msg_i, msg_f, msg_o,
             cnt_sh, sum_sh, info_i, info_f):
    cid = lax.axis_index("c")
    sid = lax.axis_index("s")
    img = cid * (B // NC) + sid // GROUP
    il = sid // GROUP
    gidx = sid % GROUP
    is_leader = gidx == 0

    _zero_hists(cnt_h, sum_h)
    _hist_pass1(img, gidx, pred_hbm, targ_hbm, mask_hbm,
                [(bp0, bt0, bm0), (bp1, bt1, bm1)], [sem0, sem1], eb_c,
                cnt_h, sum_h)
    _lane_reduce(cnt_h, sum_h, cnt_m, sum_m)
    pltpu.sync_copy(cnt_m, cnt_sh.at[sid])
    pltpu.sync_copy(sum_m, sum_sh.at[sid])
    plsc.subcore_barrier()

    @pl.when(is_leader)
    def _leader1():
        _merge_group(sid, cnt_sh, sum_sh, cnt_h, sum_h, cnt_m, sum_m)
        m_cnt = jnp.int32(0)

        def tbody(c, acc):
            return acc + jnp.sum(cnt_m[pl.ds(c * L, L)])
        m_cnt = lax.fori_loop(0, NBIN // L, tbody, m_cnt)
        vk = (jnp.full((L,), m_cnt, jnp.int32).astype(jnp.float32)
              * jnp.float32(0.8)).astype(jnp.int32)
        k = _extract(vk, 0)
        bstar, c_below, s_below = _scan_select(cnt_m, sum_m, k)
        vi = jnp.zeros((L,), jnp.int32)
        vi = _put(vi, 0, bstar)
        vi = _put(vi, 1, k)
        vi = _put(vi, 2, c_below)
        vi = _put(vi, 3, m_cnt)
        msg_i[pl.ds(0, L)] = vi
        msg_f[pl.ds(0, L)] = _put(jnp.zeros((L,), jnp.float32), 0, s_below)
        pltpu.sync_copy(msg_i, info_i.at[il])
        pltpu.sync_copy(msg_f, info_f.at[il])

    plsc.subcore_barrier()
    pltpu.sync_copy(info_i.at[il], msg_i)
    vi = msg_i[pl.ds(0, L)]
    bstar = _extract(vi, 0)

    _zero_cnt(cnt_h)
    _hist_pass2(eb_c, cnt_h, bstar)
    _lane_reduce_cnt(cnt_h, cnt_m)
    pltpu.sync_copy(cnt_m, cnt_sh.at[sid])
    plsc.subcore_barrier()

    @pl.when(is_leader)
    def _leader2():
        pltpu.sync_copy(info_f.at[il], msg_f)
        vi2 = msg_i[pl.ds(0, L)]
        k = _extract(vi2, 1)
        c_below = _extract(vi2, 2)
        m_cnt = _extract(vi2, 3)
        s_below = _extract(msg_f[pl.ds(0, L)], 0)
        _merge_group_cnt(sid, cnt_sh, cnt_h, cnt_m)
        k2 = k - c_below
        b2, c2_below, s2_mid = _scan_select_mid(cnt_m, k2, bstar)
        r = k2 - c2_below
        vbits = jnp.full((L,), lax.shift_left(bstar, 21) + jnp.int32(0x400)
                         + lax.shift_left(b2, 11), jnp.int32)
        midv = _extract(lax.bitcast_convert_type(vbits, jnp.float32), 0)
        vi_ = jnp.zeros((L,), jnp.int32)
        vi_ = _put(vi_, 2, r)
        vi_ = _put(vi_, 5, m_cnt)
        vo = vi_.astype(jnp.float32)
        vo = _put(vo, 0, s_below)
        vo = _put(vo, 1, s2_mid)
        vo = _put(vo, 4, midv)
        msg_o[...] = vo
        pltpu.sync_copy(msg_o, out_hbm.at[img])


def _finalize_kernel(x_ref, out_ref):
    x = x_ref[...]
    s_below, s2_mid = x[:, 0:1], x[:, 1:2]
    r, midv = x[:, 2:3], x[:, 4:5]
    m = x[:, 5:6]
    numer = jnp.sum(s_below + s2_mid + r * midv, axis=0, keepdims=True)
    divisor = jnp.sum(m * jnp.float32(0.8), axis=0, keepdims=True)
    out_ref[...] = jnp.where(divisor == 0.0, jnp.float32(0.0),
                             numer[:, 0:1] / jnp.maximum(divisor,
                                                         jnp.float32(1e-12)))


@jax.jit
def kernel(prediction, target, mask):
    pred = prediction.reshape(B, HW)
    targ = target.reshape(B, HW)
    mflat = mask.reshape(B, HW)

    sc = pl.kernel(
        _sc_body,
        out_type=jax.ShapeDtypeStruct((B, L), jnp.float32),
        mesh=plsc.VectorSubcoreMesh(core_axis_name="c", subcore_axis_name="s",
                                    num_cores=NC, num_subcores=NS),
        compiler_params=pltpu.CompilerParams(needs_layout_passes=False),
        scratch_types=[
            pltpu.VMEM((CH,), jnp.float32),
            pltpu.VMEM((CH,), jnp.float32),
            pltpu.VMEM((CH,), jnp.int32),
            pltpu.VMEM((CH,), jnp.float32),
            pltpu.VMEM((CH,), jnp.float32),
            pltpu.VMEM((CH,), jnp.int32),
            pltpu.SemaphoreType.DMA,
            pltpu.SemaphoreType.DMA,
            pltpu.VMEM((TPT,), jnp.int32),
            pltpu.VMEM((HLEN,), jnp.int32),
            pltpu.VMEM((HLEN,), jnp.float32),
            pltpu.VMEM((NBIN,), jnp.int32),
            pltpu.VMEM((NBIN,), jnp.float32),
            pltpu.VMEM((NBIN,), jnp.int32),
            pltpu.VMEM((NBIN,), jnp.float32),
            pltpu.VMEM((L,), jnp.float32),
            pltpu.VMEM_SHARED((NS, NBIN), jnp.int32),
            pltpu.VMEM_SHARED((NS, NBIN), jnp.float32),
            pltpu.VMEM_SHARED((B // NC, NBIN), jnp.int32),
            pltpu.VMEM_SHARED((B // NC, NBIN), jnp.float32),
        ],
    )
    per_image = sc(pred, targ, mflat)

    x = per_image
    s_below, s2_mid, r, midv, m = x[:, 0], x[:, 1], x[:, 2], x[:, 4], x[:, 5]
    numer = jnp.sum(s_below + s2_mid + r * midv)
    divisor = jnp.sum(m * jnp.float32(0.8))
    return jnp.where(divisor == 0.0, jnp.float32(0.0),
                     numer / jnp.maximum(divisor, jnp.float32(1e-12)))


# speedup vs baseline: 1.0700x; 1.0700x over previous
import functools

import jax
import jax.numpy as jnp
from jax import lax
from jax.experimental import pallas as pl
from jax.experimental.pallas import tpu as pltpu
from jax.experimental.pallas import tpu_sc as plsc

NC = 2
NS = 16
L = 16
B = 8
HW = 147456
GROUP = 4
TPT = HW // GROUP
CH = 6144
NCH = TPT // CH
NBIN = 1024
HLEN = L * NBIN


def _extract(v, j):
    return jnp.sum(jnp.where(lax.iota(jnp.int32, L) == j, v, jnp.zeros_like(v)))


def _put(v, j, val):
    return jnp.where(lax.iota(jnp.int32, L) == j, val, v)


def _zero_hists(cnt_h, sum_h):
    @plsc.parallel_loop(0, NBIN, unroll=8)
    def zbody(i):
        cnt_h[pl.ds(i * L, L)] = jnp.zeros((L,), jnp.int32)
        sum_h[pl.ds(i * L, L)] = jnp.zeros((L,), jnp.float32)


def _zero_cnt(cnt_h):
    @plsc.parallel_loop(0, NBIN, unroll=8)
    def zbody(i):
        cnt_h[pl.ds(i * L, L)] = jnp.zeros((L,), jnp.int32)


_SENT = 0x7FFFFFFF


def _hist_pass1(img, gidx, pred_hbm, targ_hbm, mask_hbm, bufs, sems, eb_c,
                cnt_h, sum_h):
    lane_off = lax.iota(jnp.int32, L) * NBIN
    ones = jnp.ones((L,), jnp.int32)
    base = gidx * TPT

    def issue(c):
        slot = c % 2
        off = base + c * CH
        bp, bt, bm = bufs[slot]
        sem = sems[slot]
        return (pltpu.async_copy(pred_hbm.at[img, pl.ds(off, CH)], bp, sem),
                pltpu.async_copy(targ_hbm.at[img, pl.ds(off, CH)], bt, sem),
                pltpu.async_copy(mask_hbm.at[img, pl.ds(off, CH)], bm, sem))

    pend = issue(0)
    for c in range(NCH):
        for d_ in pend:
            d_.wait()
        if c + 1 < NCH:
            pend = issue(c + 1)
        bp, bt, bm = bufs[c % 2]
        cbase = c * CH

        @plsc.parallel_loop(0, CH // L, unroll=4)
        def vec_body(j):
            s = pl.ds(j * L, L)
            err = jnp.abs(bp[s] - bt[s])
            valid = bm[s] > 0
            eb = lax.bitcast_convert_type(err, jnp.int32)
            eb_c[pl.ds(cbase + j * L, L)] = jnp.where(valid, eb,
                                                      jnp.int32(_SENT))
            idx = lane_off + lax.shift_right_logical(eb, 21)
            plsc.addupdate_scatter(cnt_h, [idx], ones, mask=valid)
            plsc.addupdate_scatter(sum_h, [idx], err, mask=valid)


def _hist_pass2(eb_c, cnt_h, bstar):
    lane_off = lax.iota(jnp.int32, L) * NBIN
    ones = jnp.ones((L,), jnp.int32)

    @plsc.parallel_loop(0, TPT // L, unroll=4)
    def vec_body(j):
        eb = eb_c[pl.ds(j * L, L)]
        valid = lax.shift_right_logical(eb, 21) == bstar
        idx = lane_off + jnp.bitwise_and(lax.shift_right_logical(eb, 11),
                                         jnp.int32(NBIN - 1))
        plsc.addupdate_scatter(cnt_h, [idx], ones, mask=valid)


def _lane_reduce(cnt_h, sum_h, cnt_m, sum_m):
    @plsc.parallel_loop(0, NBIN // L, unroll=2)
    def rbody(c):
        ac = jnp.zeros((L,), jnp.int32)
        asm = jnp.zeros((L,), jnp.float32)
        for r in range(L):
            s = pl.ds(r * NBIN + c * L, L)
            ac = ac + cnt_h[s]
            asm = asm + sum_h[s]
        cnt_m[pl.ds(c * L, L)] = ac
        sum_m[pl.ds(c * L, L)] = asm


def _lane_reduce_cnt(cnt_h, cnt_m):
    @plsc.parallel_loop(0, NBIN // L, unroll=2)
    def rbody(c):
        ac = jnp.zeros((L,), jnp.int32)
        for r in range(L):
            ac = ac + cnt_h[pl.ds(r * NBIN + c * L, L)]
        cnt_m[pl.ds(c * L, L)] = ac


def _merge_group_cnt(sid, cnt_sh, cnt_h, cnt_m):
    for j in range(GROUP):
        pltpu.sync_copy(cnt_sh.at[sid + j], cnt_h.at[pl.ds(j * NBIN, NBIN)])

    @plsc.parallel_loop(0, NBIN // L, unroll=4)
    def mbody(c):
        ac = jnp.zeros((L,), jnp.int32)
        for j in range(GROUP):
            ac = ac + cnt_h[pl.ds(j * NBIN + c * L, L)]
        cnt_m[pl.ds(c * L, L)] = ac


def _scan_select_mid(cnt_m, kk, bstar):
    lane = lax.iota(jnp.int32, L)
    hi = lax.shift_left(bstar, 21) + jnp.int32(0x400)

    def sbody(c, carry):
        run, nb, cb, sb = carry
        v = cnt_m[pl.ds(c * L, L)]
        bits = hi + lax.shift_left(c * L + lane, 11)
        mid = lax.bitcast_convert_type(bits, jnp.float32)
        cum = plsc.cumsum(v) + run
        m = cum < kk
        nb = nb + jnp.where(m, 1, 0).astype(jnp.int32)
        cb = cb + jnp.where(m, v, 0)
        sb = sb + jnp.where(m, v.astype(jnp.float32) * mid, jnp.float32(0.0))
        return run + jnp.sum(v), nb, cb, sb

    z_i = jnp.zeros((L,), jnp.int32)
    z_f = jnp.zeros((L,), jnp.float32)
    _, nb, cb, sb = lax.fori_loop(0, NBIN // L, sbody,
                                  (jnp.int32(0), z_i, z_i, z_f))
    return jnp.sum(nb), jnp.sum(cb), jnp.sum(sb)


def _merge_group(sid, cnt_sh, sum_sh, cnt_h, sum_h, cnt_m, sum_m):
    for j in range(GROUP):
        pltpu.sync_copy(cnt_sh.at[sid + j], cnt_h.at[pl.ds(j * NBIN, NBIN)])
        pltpu.sync_copy(sum_sh.at[sid + j], sum_h.at[pl.ds(j * NBIN, NBIN)])

    @plsc.parallel_loop(0, NBIN // L, unroll=4)
    def mbody(c):
        ac = jnp.zeros((L,), jnp.int32)
        asm = jnp.zeros((L,), jnp.float32)
        for j in range(GROUP):
            s = pl.ds(j * NBIN + c * L, L)
            ac = ac + cnt_h[s]
            asm = asm + sum_h[s]
        cnt_m[pl.ds(c * L, L)] = ac
        sum_m[pl.ds(c * L, L)] = asm


def _scan_select(cnt_m, sum_m, kk):
    def sbody(c, carry):
        run, nb, cb, sb = carry
        v = cnt_m[pl.ds(c * L, L)]
        sv = sum_m[pl.ds(c * L, L)]
        cum = plsc.cumsum(v) + run
        m = cum < kk
        nb = nb + jnp.where(m, 1, 0).astype(jnp.int32)
        cb = cb + jnp.where(m, v, 0)
        sb = sb + jnp.where(m, sv, jnp.float32(0.0))
        return run + jnp.sum(v), nb, cb, sb

    z_i = jnp.zeros((L,), jnp.int32)
    z_f = jnp.zeros((L,), jnp.float32)
    _, nb, cb, sb = lax.fori_loop(0, NBIN // L, sbody,
                                  (jnp.int32(0), z_i, z_i, z_f))
    return jnp.sum(nb), jnp.sum(cb), jnp.sum(sb)


def _bin_at(cnt_m, sum_m, b):
    lane = lax.iota(jnp.int32, L)

    def gbody(c, carry):
        ac, asm = carry
        gidx = c * L + lane
        sel = gidx == b
        ac = ac + jnp.where(sel, cnt_m[pl.ds(c * L, L)], 0)
        asm = asm + jnp.where(sel, sum_m[pl.ds(c * L, L)], jnp.float32(0.0))
        return ac, asm

    ac, asm = lax.fori_loop(0, NBIN // L, gbody,
                            (jnp.zeros((L,), jnp.int32),
                             jnp.zeros((L,), jnp.float32)))
    return jnp.sum(ac), jnp.sum(asm)


def _sc_body(pred_hbm, targ_hbm, mask_hbm, out_hbm,
             bp0, bt0, bm0, bp1, bt1, bm1, sem0, sem1, eb_c,
             cnt_h, sum_h, cnt_m, sum_m, msg_i, msg_f, msg_o,
             cnt_sh, sum_sh, info_i, info_f):
    cid = lax.axis_index("c")
    sid = lax.axis_index("s")
    img = cid * (B // NC) + sid // GROUP
    il = sid // GROUP
    gidx = sid % GROUP
    is_leader = gidx == 0

    _zero_hists(cnt_h, sum_h)
    _hist_pass1(img, gidx, pred_hbm, targ_hbm, mask_hbm,
                [(bp0, bt0, bm0), (bp1, bt1, bm1)], [sem0, sem1], eb_c,
                cnt_h, sum_h)
    _lane_reduce(cnt_h, sum_h, cnt_m, sum_m)
    pltpu.sync_copy(cnt_m, cnt_sh.at[sid])
    pltpu.sync_copy(sum_m, sum_sh.at[sid])
    plsc.subcore_barrier()

    @pl.when(is_leader)
    def _leader1():
        _merge_group(sid, cnt_sh, sum_sh, cnt_h, sum_h, cnt_m, sum_m)
        m_cnt = jnp.int32(0)

        def tbody(c, acc):
            return acc + jnp.sum(cnt_m[pl.ds(c * L, L)])
        m_cnt = lax.fori_loop(0, NBIN // L, tbody, m_cnt)
        vk = (jnp.full((L,), m_cnt, jnp.int32).astype(jnp.float32)
              * jnp.float32(0.8)).astype(jnp.int32)
        k = _extract(vk, 0)
        bstar, c_below, s_below = _scan_select(cnt_m, sum_m, k)
        vi = jnp.zeros((L,), jnp.int32)
        vi = _put(vi, 0, bstar)
        vi = _put(vi, 1, k)
        vi = _put(vi, 2, c_below)
        vi = _put(vi, 3, m_cnt)
        msg_i[pl.ds(0, L)] = vi
        msg_f[pl.ds(0, L)] = _put(jnp.zeros((L,), jnp.float32), 0, s_below)
        pltpu.sync_copy(msg_i, info_i.at[il])
        pltpu.sync_copy(msg_f, info_f.at[il])

    plsc.subcore_barrier()
    pltpu.sync_copy(info_i.at[il], msg_i)
    vi = msg_i[pl.ds(0, L)]
    bstar = _extract(vi, 0)

    _zero_cnt(cnt_h)
    _hist_pass2(eb_c, cnt_h, bstar)
    _lane_reduce_cnt(cnt_h, cnt_m)
    pltpu.sync_copy(cnt_m, cnt_sh.at[sid])
    plsc.subcore_barrier()

    @pl.when(is_leader)
    def _leader2():
        pltpu.sync_copy(info_f.at[il], msg_f)
        vi2 = msg_i[pl.ds(0, L)]
        k = _extract(vi2, 1)
        c_below = _extract(vi2, 2)
        m_cnt = _extract(vi2, 3)
        s_below = _extract(msg_f[pl.ds(0, L)], 0)
        _merge_group_cnt(sid, cnt_sh, cnt_h, cnt_m)
        k2 = k - c_below
        b2, c2_below, s2_mid = _scan_select_mid(cnt_m, k2, bstar)
        r = k2 - c2_below
        vbits = jnp.full((L,), lax.shift_left(bstar, 21) + jnp.int32(0x400)
                         + lax.shift_left(b2, 11), jnp.int32)
        midv = _extract(lax.bitcast_convert_type(vbits, jnp.float32), 0)
        vi_ = jnp.zeros((L,), jnp.int32)
        vi_ = _put(vi_, 2, r)
        vi_ = _put(vi_, 5, m_cnt)
        vo = vi_.astype(jnp.float32)
        vo = _put(vo, 0, s_below)
        vo = _put(vo, 1, s2_mid)
        vo = _put(vo, 4, midv)
        msg_o[...] = vo
        pltpu.sync_copy(msg_o, out_hbm.at[img])


def _finalize_kernel(x_ref, out_ref):
    x = x_ref[...]
    s_below, s2_mid = x[:, 0:1], x[:, 1:2]
    r, midv = x[:, 2:3], x[:, 4:5]
    m = x[:, 5:6]
    numer = jnp.sum(s_below + s2_mid + r * midv, axis=0, keepdims=True)
    divisor = jnp.sum(m * jnp.float32(0.8), axis=0, keepdims=True)
    out_ref[...] = jnp.where(divisor == 0.0, jnp.float32(0.0),
                             numer[:, 0:1] / jnp.maximum(divisor,
                                                         jnp.float32(1e-12)))


@jax.jit
def kernel(prediction, target, mask):
    pred = prediction.reshape(B, HW)
    targ = target.reshape(B, HW)
    mflat = mask.reshape(B, HW)

    sc = pl.kernel(
        _sc_body,
        out_type=jax.ShapeDtypeStruct((B, L), jnp.float32),
        mesh=plsc.VectorSubcoreMesh(core_axis_name="c", subcore_axis_name="s",
                                    num_cores=NC, num_subcores=NS),
        compiler_params=pltpu.CompilerParams(needs_layout_passes=False, skip_device_barrier=True),
        scratch_types=[
            pltpu.VMEM((CH,), jnp.float32),
            pltpu.VMEM((CH,), jnp.float32),
            pltpu.VMEM((CH,), jnp.int32),
            pltpu.VMEM((CH,), jnp.float32),
            pltpu.VMEM((CH,), jnp.float32),
            pltpu.VMEM((CH,), jnp.int32),
            pltpu.SemaphoreType.DMA,
            pltpu.SemaphoreType.DMA,
            pltpu.VMEM((TPT,), jnp.int32),
            pltpu.VMEM((HLEN,), jnp.int32),
            pltpu.VMEM((HLEN,), jnp.float32),
            pltpu.VMEM((NBIN,), jnp.int32),
            pltpu.VMEM((NBIN,), jnp.float32),
            pltpu.VMEM((NBIN,), jnp.int32),
            pltpu.VMEM((NBIN,), jnp.float32),
            pltpu.VMEM((L,), jnp.float32),
            pltpu.VMEM_SHARED((NS, NBIN), jnp.int32),
            pltpu.VMEM_SHARED((NS, NBIN), jnp.float32),
            pltpu.VMEM_SHARED((B // NC, NBIN), jnp.int32),
            pltpu.VMEM_SHARED((B // NC, NBIN), jnp.float32),
        ],
    )
    per_image = sc(pred, targ, mflat)

    out = pl.pallas_call(
        _finalize_kernel,
        out_shape=jax.ShapeDtypeStruct((1, 1), jnp.float32),
    )(per_image)
    return out.reshape(())


# speedup vs baseline: 1.0797x; 1.0091x over previous
import functools

import jax
import jax.numpy as jnp
from jax import lax
from jax.experimental import pallas as pl
from jax.experimental.pallas import tpu as pltpu
from jax.experimental.pallas import tpu_sc as plsc

NC = 2
NS = 16
L = 16
B = 8
HW = 147456
GROUP = 4
TPT = HW // GROUP
CH = 6144
NCH = TPT // CH
NBIN = 1024
HLEN = L * NBIN


def _extract(v, j):
    return jnp.sum(jnp.where(lax.iota(jnp.int32, L) == j, v, jnp.zeros_like(v)))


def _put(v, j, val):
    return jnp.where(lax.iota(jnp.int32, L) == j, val, v)


def _zero_hists(cnt_h, sum_h):
    @plsc.parallel_loop(0, NBIN, unroll=8)
    def zbody(i):
        cnt_h[pl.ds(i * L, L)] = jnp.zeros((L,), jnp.int32)
        sum_h[pl.ds(i * L, L)] = jnp.zeros((L,), jnp.float32)


def _zero_cnt(cnt_h):
    @plsc.parallel_loop(0, NBIN, unroll=8)
    def zbody(i):
        cnt_h[pl.ds(i * L, L)] = jnp.zeros((L,), jnp.int32)


_SENT = 0x7FFFFFFF


def _hist_pass1(img, gidx, pred_hbm, targ_hbm, mask_hbm, bufs, sems, eb_c,
                cnt_h, sum_h):
    lane_off = lax.iota(jnp.int32, L) * NBIN
    ones = jnp.ones((L,), jnp.int32)
    base = gidx * TPT

    def issue(c):
        slot = c % 2
        off = base + c * CH
        bp, bt, bm = bufs[slot]
        sem = sems[slot]
        return (pltpu.async_copy(pred_hbm.at[img, pl.ds(off, CH)], bp, sem),
                pltpu.async_copy(targ_hbm.at[img, pl.ds(off, CH)], bt, sem),
                pltpu.async_copy(mask_hbm.at[img, pl.ds(off, CH)], bm, sem))

    pend = issue(0)
    for c in range(NCH):
        for d_ in pend:
            d_.wait()
        if c + 1 < NCH:
            pend = issue(c + 1)
        bp, bt, bm = bufs[c % 2]
        cbase = c * CH

        @plsc.parallel_loop(0, CH // L, unroll=4)
        def vec_body(j):
            s = pl.ds(j * L, L)
            err = jnp.abs(bp[s] - bt[s])
            valid = bm[s] > 0
            eb = lax.bitcast_convert_type(err, jnp.int32)
            eb_c[pl.ds(cbase + j * L, L)] = jnp.where(valid, eb,
                                                      jnp.int32(_SENT))
            idx = lane_off + lax.shift_right_logical(eb, 21)
            plsc.addupdate_scatter(cnt_h, [idx], ones, mask=valid)
            plsc.addupdate_scatter(sum_h, [idx], err, mask=valid)


def _hist_pass2(eb_c, cnt_h, bstar):
    lane_off = lax.iota(jnp.int32, L) * NBIN
    ones = jnp.ones((L,), jnp.int32)

    @plsc.parallel_loop(0, TPT // L, unroll=4)
    def vec_body(j):
        eb = eb_c[pl.ds(j * L, L)]
        valid = lax.shift_right_logical(eb, 21) == bstar
        idx = lane_off + jnp.bitwise_and(lax.shift_right_logical(eb, 11),
                                         jnp.int32(NBIN - 1))
        plsc.addupdate_scatter(cnt_h, [idx], ones, mask=valid)


def _lane_reduce(cnt_h, sum_h, cnt_m, sum_m):
    @plsc.parallel_loop(0, NBIN // L, unroll=2)
    def rbody(c):
        ac = jnp.zeros((L,), jnp.int32)
        asm = jnp.zeros((L,), jnp.float32)
        for r in range(L):
            s = pl.ds(r * NBIN + c * L, L)
            ac = ac + cnt_h[s]
            asm = asm + sum_h[s]
        cnt_m[pl.ds(c * L, L)] = ac
        sum_m[pl.ds(c * L, L)] = asm


def _lane_reduce_cnt(cnt_h, cnt_m):
    @plsc.parallel_loop(0, NBIN // L, unroll=2)
    def rbody(c):
        ac = jnp.zeros((L,), jnp.int32)
        for r in range(L):
            ac = ac + cnt_h[pl.ds(r * NBIN + c * L, L)]
        cnt_m[pl.ds(c * L, L)] = ac


def _merge_group_cnt(sid, cnt_sh, cnt_h, cnt_m):
    for j in range(GROUP):
        pltpu.sync_copy(cnt_sh.at[sid + j], cnt_h.at[pl.ds(j * NBIN, NBIN)])

    @plsc.parallel_loop(0, NBIN // L, unroll=4)
    def mbody(c):
        ac = jnp.zeros((L,), jnp.int32)
        for j in range(GROUP):
            ac = ac + cnt_h[pl.ds(j * NBIN + c * L, L)]
        cnt_m[pl.ds(c * L, L)] = ac


def _scan_select_mid(cnt_m, kk, bstar):
    lane = lax.iota(jnp.int32, L)
    hi = lax.shift_left(bstar, 21) + jnp.int32(0x400)

    def sbody(c, carry):
        run, nb, cb, sb = carry
        v = cnt_m[pl.ds(c * L, L)]
        bits = hi + lax.shift_left(c * L + lane, 11)
        mid = lax.bitcast_convert_type(bits, jnp.float32)
        cum = plsc.cumsum(v) + run
        m = cum < kk
        nb = nb + jnp.where(m, 1, 0).astype(jnp.int32)
        cb = cb + jnp.where(m, v, 0)
        sb = sb + jnp.where(m, v.astype(jnp.float32) * mid, jnp.float32(0.0))
        return run + jnp.sum(v), nb, cb, sb

    z_i = jnp.zeros((L,), jnp.int32)
    z_f = jnp.zeros((L,), jnp.float32)
    _, nb, cb, sb = lax.fori_loop(0, NBIN // L, sbody,
                                  (jnp.int32(0), z_i, z_i, z_f))
    return jnp.sum(nb), jnp.sum(cb), jnp.sum(sb)


def _merge_group(sid, cnt_sh, sum_sh, cnt_h, sum_h, cnt_m, sum_m):
    for j in range(GROUP):
        pltpu.sync_copy(cnt_sh.at[sid + j], cnt_h.at[pl.ds(j * NBIN, NBIN)])
        pltpu.sync_copy(sum_sh.at[sid + j], sum_h.at[pl.ds(j * NBIN, NBIN)])

    @plsc.parallel_loop(0, NBIN // L, unroll=4)
    def mbody(c):
        ac = jnp.zeros((L,), jnp.int32)
        asm = jnp.zeros((L,), jnp.float32)
        for j in range(GROUP):
            s = pl.ds(j * NBIN + c * L, L)
            ac = ac + cnt_h[s]
            asm = asm + sum_h[s]
        cnt_m[pl.ds(c * L, L)] = ac
        sum_m[pl.ds(c * L, L)] = asm


def _scan_select(cnt_m, sum_m, kk):
    def sbody(c, carry):
        run, nb, cb, sb = carry
        v = cnt_m[pl.ds(c * L, L)]
        sv = sum_m[pl.ds(c * L, L)]
        cum = plsc.cumsum(v) + run
        m = cum < kk
        nb = nb + jnp.where(m, 1, 0).astype(jnp.int32)
        cb = cb + jnp.where(m, v, 0)
        sb = sb + jnp.where(m, sv, jnp.float32(0.0))
        return run + jnp.sum(v), nb, cb, sb

    z_i = jnp.zeros((L,), jnp.int32)
    z_f = jnp.zeros((L,), jnp.float32)
    _, nb, cb, sb = lax.fori_loop(0, NBIN // L, sbody,
                                  (jnp.int32(0), z_i, z_i, z_f))
    return jnp.sum(nb), jnp.sum(cb), jnp.sum(sb)


def _bin_at(cnt_m, sum_m, b):
    lane = lax.iota(jnp.int32, L)

    def gbody(c, carry):
        ac, asm = carry
        gidx = c * L + lane
        sel = gidx == b
        ac = ac + jnp.where(sel, cnt_m[pl.ds(c * L, L)], 0)
        asm = asm + jnp.where(sel, sum_m[pl.ds(c * L, L)], jnp.float32(0.0))
        return ac, asm

    ac, asm = lax.fori_loop(0, NBIN // L, gbody,
                            (jnp.zeros((L,), jnp.int32),
                             jnp.zeros((L,), jnp.float32)))
    return jnp.sum(ac), jnp.sum(asm)


def _sc_body(pred_hbm, targ_hbm, mask_hbm, out_hbm,
             bp0, bt0, bm0, bp1, bt1, bm1, sem0, sem1, eb_c,
             cnt_h, sum_h, cnt_m, sum_m, msg_o,
             cnt_sh, sum_sh):
    cid = lax.axis_index("c")
    sid = lax.axis_index("s")
    img = cid * (B // NC) + sid // GROUP
    il = sid // GROUP
    gidx = sid % GROUP
    is_leader = gidx == 0

    _zero_hists(cnt_h, sum_h)
    _hist_pass1(img, gidx, pred_hbm, targ_hbm, mask_hbm,
                [(bp0, bt0, bm0), (bp1, bt1, bm1)], [sem0, sem1], eb_c,
                cnt_h, sum_h)
    _lane_reduce(cnt_h, sum_h, cnt_m, sum_m)
    pltpu.sync_copy(cnt_m, cnt_sh.at[sid])
    pltpu.sync_copy(sum_m, sum_sh.at[sid])
    plsc.subcore_barrier()

    grp = (sid // GROUP) * GROUP
    _merge_group(grp, cnt_sh, sum_sh, cnt_h, sum_h, cnt_m, sum_m)

    def tbody(c, acc):
        return acc + jnp.sum(cnt_m[pl.ds(c * L, L)])
    m_cnt = lax.fori_loop(0, NBIN // L, tbody, jnp.int32(0))
    vk = (jnp.full((L,), m_cnt, jnp.int32).astype(jnp.float32)
          * jnp.float32(0.8)).astype(jnp.int32)
    k = _extract(vk, 0)
    bstar, c_below, s_below = _scan_select(cnt_m, sum_m, k)

    _zero_cnt(cnt_h)
    _hist_pass2(eb_c, cnt_h, bstar)
    _lane_reduce_cnt(cnt_h, cnt_m)
    pltpu.sync_copy(cnt_m, cnt_sh.at[sid])
    plsc.subcore_barrier()

    @pl.when(is_leader)
    def _leader2():
        _merge_group_cnt(sid, cnt_sh, cnt_h, cnt_m)
        k2 = k - c_below
        b2, c2_below, s2_mid = _scan_select_mid(cnt_m, k2, bstar)
        r = k2 - c2_below
        vbits = jnp.full((L,), lax.shift_left(bstar, 21) + jnp.int32(0x400)
                         + lax.shift_left(b2, 11), jnp.int32)
        midv = _extract(lax.bitcast_convert_type(vbits, jnp.float32), 0)
        vi_ = jnp.zeros((L,), jnp.int32)
        vi_ = _put(vi_, 2, r)
        vi_ = _put(vi_, 5, m_cnt)
        vo = vi_.astype(jnp.float32)
        vo = _put(vo, 0, s_below)
        vo = _put(vo, 1, s2_mid)
        vo = _put(vo, 4, midv)
        msg_o[...] = vo
        pltpu.sync_copy(msg_o, out_hbm.at[img])


def _finalize_kernel(x_ref, out_ref):
    x = x_ref[...]
    s_below, s2_mid = x[:, 0:1], x[:, 1:2]
    r, midv = x[:, 2:3], x[:, 4:5]
    m = x[:, 5:6]
    numer = jnp.sum(s_below + s2_mid + r * midv, axis=0, keepdims=True)
    divisor = jnp.sum(m * jnp.float32(0.8), axis=0, keepdims=True)
    out_ref[...] = jnp.where(divisor == 0.0, jnp.float32(0.0),
                             numer[:, 0:1] / jnp.maximum(divisor,
                                                         jnp.float32(1e-12)))


@jax.jit
def kernel(prediction, target, mask):
    pred = prediction.reshape(B, HW)
    targ = target.reshape(B, HW)
    mflat = mask.reshape(B, HW)

    sc = pl.kernel(
        _sc_body,
        out_type=jax.ShapeDtypeStruct((B, L), jnp.float32),
        mesh=plsc.VectorSubcoreMesh(core_axis_name="c", subcore_axis_name="s",
                                    num_cores=NC, num_subcores=NS),
        compiler_params=pltpu.CompilerParams(needs_layout_passes=False, skip_device_barrier=True),
        scratch_types=[
            pltpu.VMEM((CH,), jnp.float32),
            pltpu.VMEM((CH,), jnp.float32),
            pltpu.VMEM((CH,), jnp.int32),
            pltpu.VMEM((CH,), jnp.float32),
            pltpu.VMEM((CH,), jnp.float32),
            pltpu.VMEM((CH,), jnp.int32),
            pltpu.SemaphoreType.DMA,
            pltpu.SemaphoreType.DMA,
            pltpu.VMEM((TPT,), jnp.int32),
            pltpu.VMEM((HLEN,), jnp.int32),
            pltpu.VMEM((HLEN,), jnp.float32),
            pltpu.VMEM((NBIN,), jnp.int32),
            pltpu.VMEM((NBIN,), jnp.float32),
            pltpu.VMEM((L,), jnp.float32),
            pltpu.VMEM_SHARED((NS, NBIN), jnp.int32),
            pltpu.VMEM_SHARED((NS, NBIN), jnp.float32),
        ],
    )
    per_image = sc(pred, targ, mflat)

    out = pl.pallas_call(
        _finalize_kernel,
        out_shape=jax.ShapeDtypeStruct((1, 1), jnp.float32),
    )(per_image)
    return out.reshape(())


# speedup vs baseline: 1.0890x; 1.0086x over previous
import functools

import jax
import jax.numpy as jnp
from jax import lax
from jax.experimental import pallas as pl
from jax.experimental.pallas import tpu as pltpu
from jax.experimental.pallas import tpu_sc as plsc

NC = 2
NS = 16
L = 16
B = 8
HW = 147456
GROUP = 4
TPT = HW // GROUP
CH = 9216
NCH = TPT // CH
NBIN = 1024
HLEN = L * NBIN


def _extract(v, j):
    return jnp.sum(jnp.where(lax.iota(jnp.int32, L) == j, v, jnp.zeros_like(v)))


def _put(v, j, val):
    return jnp.where(lax.iota(jnp.int32, L) == j, val, v)


def _zero_hists(cnt_h, sum_h):
    @plsc.parallel_loop(0, NBIN, unroll=8)
    def zbody(i):
        cnt_h[pl.ds(i * L, L)] = jnp.zeros((L,), jnp.int32)
        sum_h[pl.ds(i * L, L)] = jnp.zeros((L,), jnp.float32)


def _zero_cnt(cnt_h):
    @plsc.parallel_loop(0, NBIN, unroll=8)
    def zbody(i):
        cnt_h[pl.ds(i * L, L)] = jnp.zeros((L,), jnp.int32)


_SENT = 0x7FFFFFFF


def _hist_pass1(img, gidx, pred_hbm, targ_hbm, mask_hbm, bufs, sems, eb_c,
                cnt_h, sum_h):
    lane_off = lax.iota(jnp.int32, L) * NBIN
    ones = jnp.ones((L,), jnp.int32)
    base = gidx * TPT

    def issue(c):
        slot = c % 2
        off = base + c * CH
        bp, bt, bm = bufs[slot]
        sem = sems[slot]
        return (pltpu.async_copy(pred_hbm.at[img, pl.ds(off, CH)], bp, sem),
                pltpu.async_copy(targ_hbm.at[img, pl.ds(off, CH)], bt, sem),
                pltpu.async_copy(mask_hbm.at[img, pl.ds(off, CH)], bm, sem))

    pend = issue(0)
    for c in range(NCH):
        for d_ in pend:
            d_.wait()
        if c + 1 < NCH:
            pend = issue(c + 1)
        bp, bt, bm = bufs[c % 2]
        cbase = c * CH

        @plsc.parallel_loop(0, CH // L, unroll=4)
        def vec_body(j):
            s = pl.ds(j * L, L)
            err = jnp.abs(bp[s] - bt[s])
            valid = bm[s] > 0
            eb = lax.bitcast_convert_type(err, jnp.int32)
            eb_c[pl.ds(cbase + j * L, L)] = jnp.where(valid, eb,
                                                      jnp.int32(_SENT))
            idx = lane_off + lax.shift_right_logical(eb, 21)
            plsc.addupdate_scatter(cnt_h, [idx], ones, mask=valid)
            plsc.addupdate_scatter(sum_h, [idx], err, mask=valid)


def _hist_pass2(eb_c, cnt_h, bstar):
    lane_off = lax.iota(jnp.int32, L) * NBIN
    ones = jnp.ones((L,), jnp.int32)

    @plsc.parallel_loop(0, TPT // L, unroll=4)
    def vec_body(j):
        eb = eb_c[pl.ds(j * L, L)]
        valid = lax.shift_right_logical(eb, 21) == bstar
        idx = lane_off + jnp.bitwise_and(lax.shift_right_logical(eb, 11),
                                         jnp.int32(NBIN - 1))
        plsc.addupdate_scatter(cnt_h, [idx], ones, mask=valid)


def _lane_reduce(cnt_h, sum_h, cnt_m, sum_m):
    @plsc.parallel_loop(0, NBIN // L, unroll=2)
    def rbody(c):
        ac = jnp.zeros((L,), jnp.int32)
        asm = jnp.zeros((L,), jnp.float32)
        for r in range(L):
            s = pl.ds(r * NBIN + c * L, L)
            ac = ac + cnt_h[s]
            asm = asm + sum_h[s]
        cnt_m[pl.ds(c * L, L)] = ac
        sum_m[pl.ds(c * L, L)] = asm


def _lane_reduce_cnt(cnt_h, cnt_m):
    @plsc.parallel_loop(0, NBIN // L, unroll=2)
    def rbody(c):
        ac = jnp.zeros((L,), jnp.int32)
        for r in range(L):
            ac = ac + cnt_h[pl.ds(r * NBIN + c * L, L)]
        cnt_m[pl.ds(c * L, L)] = ac


def _merge_group_cnt(sid, cnt_sh, cnt_h, cnt_m):
    for j in range(GROUP):
        pltpu.sync_copy(cnt_sh.at[sid + j], cnt_h.at[pl.ds(j * NBIN, NBIN)])

    @plsc.parallel_loop(0, NBIN // L, unroll=4)
    def mbody(c):
        ac = jnp.zeros((L,), jnp.int32)
        for j in range(GROUP):
            ac = ac + cnt_h[pl.ds(j * NBIN + c * L, L)]
        cnt_m[pl.ds(c * L, L)] = ac


def _scan_select_mid(cnt_m, kk, bstar):
    lane = lax.iota(jnp.int32, L)
    hi = lax.shift_left(bstar, 21) + jnp.int32(0x400)

    def sbody(c, carry):
        run, nb, cb, sb = carry
        v = cnt_m[pl.ds(c * L, L)]
        bits = hi + lax.shift_left(c * L + lane, 11)
        mid = lax.bitcast_convert_type(bits, jnp.float32)
        cum = plsc.cumsum(v) + run
        m = cum < kk
        nb = nb + jnp.where(m, 1, 0).astype(jnp.int32)
        cb = cb + jnp.where(m, v, 0)
        sb = sb + jnp.where(m, v.astype(jnp.float32) * mid, jnp.float32(0.0))
        return run + jnp.sum(v), nb, cb, sb

    z_i = jnp.zeros((L,), jnp.int32)
    z_f = jnp.zeros((L,), jnp.float32)
    _, nb, cb, sb = lax.fori_loop(0, NBIN // L, sbody,
                                  (jnp.int32(0), z_i, z_i, z_f))
    return jnp.sum(nb), jnp.sum(cb), jnp.sum(sb)


def _merge_group(sid, cnt_sh, sum_sh, cnt_h, sum_h, cnt_m, sum_m):
    for j in range(GROUP):
        pltpu.sync_copy(cnt_sh.at[sid + j], cnt_h.at[pl.ds(j * NBIN, NBIN)])
        pltpu.sync_copy(sum_sh.at[sid + j], sum_h.at[pl.ds(j * NBIN, NBIN)])

    @plsc.parallel_loop(0, NBIN // L, unroll=4)
    def mbody(c):
        ac = jnp.zeros((L,), jnp.int32)
        asm = jnp.zeros((L,), jnp.float32)
        for j in range(GROUP):
            s = pl.ds(j * NBIN + c * L, L)
            ac = ac + cnt_h[s]
            asm = asm + sum_h[s]
        cnt_m[pl.ds(c * L, L)] = ac
        sum_m[pl.ds(c * L, L)] = asm


def _scan_select(cnt_m, sum_m, kk):
    def sbody(c, carry):
        run, nb, cb, sb = carry
        v = cnt_m[pl.ds(c * L, L)]
        sv = sum_m[pl.ds(c * L, L)]
        cum = plsc.cumsum(v) + run
        m = cum < kk
        nb = nb + jnp.where(m, 1, 0).astype(jnp.int32)
        cb = cb + jnp.where(m, v, 0)
        sb = sb + jnp.where(m, sv, jnp.float32(0.0))
        return run + jnp.sum(v), nb, cb, sb

    z_i = jnp.zeros((L,), jnp.int32)
    z_f = jnp.zeros((L,), jnp.float32)
    _, nb, cb, sb = lax.fori_loop(0, NBIN // L, sbody,
                                  (jnp.int32(0), z_i, z_i, z_f))
    return jnp.sum(nb), jnp.sum(cb), jnp.sum(sb)


def _bin_at(cnt_m, sum_m, b):
    lane = lax.iota(jnp.int32, L)

    def gbody(c, carry):
        ac, asm = carry
        gidx = c * L + lane
        sel = gidx == b
        ac = ac + jnp.where(sel, cnt_m[pl.ds(c * L, L)], 0)
        asm = asm + jnp.where(sel, sum_m[pl.ds(c * L, L)], jnp.float32(0.0))
        return ac, asm

    ac, asm = lax.fori_loop(0, NBIN // L, gbody,
                            (jnp.zeros((L,), jnp.int32),
                             jnp.zeros((L,), jnp.float32)))
    return jnp.sum(ac), jnp.sum(asm)


def _sc_body(pred_hbm, targ_hbm, mask_hbm, out_hbm,
             bp0, bt0, bm0, bp1, bt1, bm1, sem0, sem1, eb_c,
             cnt_h, sum_h, cnt_m, sum_m, msg_o,
             cnt_sh, sum_sh):
    cid = lax.axis_index("c")
    sid = lax.axis_index("s")
    img = cid * (B // NC) + sid // GROUP
    il = sid // GROUP
    gidx = sid % GROUP
    is_leader = gidx == 0

    _zero_hists(cnt_h, sum_h)
    _hist_pass1(img, gidx, pred_hbm, targ_hbm, mask_hbm,
                [(bp0, bt0, bm0), (bp1, bt1, bm1)], [sem0, sem1], eb_c,
                cnt_h, sum_h)
    _lane_reduce(cnt_h, sum_h, cnt_m, sum_m)
    pltpu.sync_copy(cnt_m, cnt_sh.at[sid])
    pltpu.sync_copy(sum_m, sum_sh.at[sid])
    plsc.subcore_barrier()

    grp = (sid // GROUP) * GROUP
    _merge_group(grp, cnt_sh, sum_sh, cnt_h, sum_h, cnt_m, sum_m)

    def tbody(c, acc):
        return acc + jnp.sum(cnt_m[pl.ds(c * L, L)])
    m_cnt = lax.fori_loop(0, NBIN // L, tbody, jnp.int32(0))
    vk = (jnp.full((L,), m_cnt, jnp.int32).astype(jnp.float32)
          * jnp.float32(0.8)).astype(jnp.int32)
    k = _extract(vk, 0)
    bstar, c_below, s_below = _scan_select(cnt_m, sum_m, k)

    _zero_cnt(cnt_h)
    _hist_pass2(eb_c, cnt_h, bstar)
    _lane_reduce_cnt(cnt_h, cnt_m)
    pltpu.sync_copy(cnt_m, cnt_sh.at[sid])
    plsc.subcore_barrier()

    @pl.when(is_leader)
    def _leader2():
        _merge_group_cnt(sid, cnt_sh, cnt_h, cnt_m)
        k2 = k - c_below
        b2, c2_below, s2_mid = _scan_select_mid(cnt_m, k2, bstar)
        r = k2 - c2_below
        vbits = jnp.full((L,), lax.shift_left(bstar, 21) + jnp.int32(0x400)
                         + lax.shift_left(b2, 11), jnp.int32)
        midv = _extract(lax.bitcast_convert_type(vbits, jnp.float32), 0)
        vi_ = jnp.zeros((L,), jnp.int32)
        vi_ = _put(vi_, 2, r)
        vi_ = _put(vi_, 5, m_cnt)
        vo = vi_.astype(jnp.float32)
        vo = _put(vo, 0, s_below)
        vo = _put(vo, 1, s2_mid)
        vo = _put(vo, 4, midv)
        msg_o[...] = vo
        pltpu.sync_copy(msg_o, out_hbm.at[img])


def _finalize_kernel(x_ref, out_ref):
    x = x_ref[...]
    s_below, s2_mid = x[:, 0:1], x[:, 1:2]
    r, midv = x[:, 2:3], x[:, 4:5]
    m = x[:, 5:6]
    numer = jnp.sum(s_below + s2_mid + r * midv, axis=0, keepdims=True)
    divisor = jnp.sum(m * jnp.float32(0.8), axis=0, keepdims=True)
    out_ref[...] = jnp.where(divisor == 0.0, jnp.float32(0.0),
                             numer[:, 0:1] / jnp.maximum(divisor,
                                                         jnp.float32(1e-12)))


@jax.jit
def kernel(prediction, target, mask):
    pred = prediction.reshape(B, HW)
    targ = target.reshape(B, HW)
    mflat = mask.reshape(B, HW)

    sc = pl.kernel(
        _sc_body,
        out_type=jax.ShapeDtypeStruct((B, L), jnp.float32),
        mesh=plsc.VectorSubcoreMesh(core_axis_name="c", subcore_axis_name="s",
                                    num_cores=NC, num_subcores=NS),
        compiler_params=pltpu.CompilerParams(needs_layout_passes=False, skip_device_barrier=True),
        scratch_types=[
            pltpu.VMEM((CH,), jnp.float32),
            pltpu.VMEM((CH,), jnp.float32),
            pltpu.VMEM((CH,), jnp.int32),
            pltpu.VMEM((CH,), jnp.float32),
            pltpu.VMEM((CH,), jnp.float32),
            pltpu.VMEM((CH,), jnp.int32),
            pltpu.SemaphoreType.DMA,
            pltpu.SemaphoreType.DMA,
            pltpu.VMEM((TPT,), jnp.int32),
            pltpu.VMEM((HLEN,), jnp.int32),
            pltpu.VMEM((HLEN,), jnp.float32),
            pltpu.VMEM((NBIN,), jnp.int32),
            pltpu.VMEM((NBIN,), jnp.float32),
            pltpu.VMEM((L,), jnp.float32),
            pltpu.VMEM_SHARED((NS, NBIN), jnp.int32),
            pltpu.VMEM_SHARED((NS, NBIN), jnp.float32),
        ],
    )
    per_image = sc(pred, targ, mflat)

    out = pl.pallas_call(
        _finalize_kernel,
        out_shape=jax.ShapeDtypeStruct((1, 1), jnp.float32),
    )(per_image)
    return out.reshape(())
